# baseline (speedup 1.0000x reference)
import functools

import jax
import jax.numpy as jnp
from jax import lax
from jax.experimental import pallas as pl
from jax.experimental.pallas import tpu as pltpu
from jax.experimental.pallas import tpu_sc as plsc

B, N = 64, 1024
B_TC = 36
B_SC = B - B_TC
NH = 24
RPC = 16
C = N // RPC
NC, NS = 2, 16
NW = NC * NS
CCW = C // NW


def _sc_body(cnt_hbm, start_hbm, order_hbm, w_hbm, imf_hbm, out_hbm,
             cnt_v, start_v, order_v, w_v, in_buf, out_buf,
             in_sems, out_sems, prev_smem):
    wid = lax.axis_index("s") * NC + lax.axis_index("c")

    def read_chunk(h, hh, cc):
        r0 = (cc * NW + wid) * RPC
        pltpu.async_copy(imf_hbm.at[h, pl.ds(r0, RPC), :],
                         in_buf.at[hh * CCW + cc], in_sems.at[hh * CCW + cc])

    for hh in range(2):
        for cc in range(CCW):
            read_chunk(hh, hh, cc)

    pltpu.sync_copy(cnt_hbm, cnt_v)
    pltpu.sync_copy(start_hbm, start_v)
    pltpu.sync_copy(order_hbm, order_v)
    pltpu.sync_copy(w_hbm, w_v)
    for cc in range(CCW):
        prev_smem[cc] = 0

    def _sget(ref, i):
        return ref[pl.ds(i, 16)][0]

    @pl.loop(0, NH, step=2)
    def _hloop(g):
        for hh in range(2):
            h = g + hh
            wv = jnp.maximum(_sget(w_v, h), 0.0)
            cnt = _sget(cnt_v, h)
            st = _sget(start_v, h)
            for cc in range(CCW):
                slot = hh * CCW + cc
                r0 = (cc * NW + wid) * RPC

                pltpu.make_async_copy(imf_hbm.at[0, pl.ds(0, RPC), :],
                                      in_buf.at[slot], in_sems.at[slot]).wait()

                prev = prev_smem[cc]

                @pl.loop(0, prev)
                def _drain(j):
                    pltpu.make_async_copy(out_buf.at[cc],
                                          out_hbm.at[0, pl.ds(0, RPC), :],
                                          out_sems.at[cc]).wait()

                for r in range(RPC):

                    @plsc.parallel_loop(0, N, step=16, unroll=16)
                    def _scale(j):
                        sl = pl.ds(j, 16)
                        out_buf[cc, r, sl] = in_buf[slot, r, sl] * wv

                @pl.when(h + 2 < NH)
                def _prefetch():
                    read_chunk(h + 2, hh, cc)

                @pl.loop(0, cnt)
                def _writes(j):
                    b = _sget(order_v, st + j)
                    pltpu.async_copy(out_buf.at[cc],
                                     out_hbm.at[b, pl.ds(r0, RPC), :],
                                     out_sems.at[cc])

                prev_smem[cc] = cnt

    for cc in range(CCW):
        prev = prev_smem[cc]

        @pl.loop(0, prev)
        def _final_drain(j):
            pltpu.make_async_copy(out_buf.at[cc],
                                  out_hbm.at[0, pl.ds(0, RPC), :],
                                  out_sems.at[cc]).wait()


_sc_call = functools.partial(
    pl.kernel,
    out_type=jax.ShapeDtypeStruct((B_SC, N, N), jnp.float32),
    mesh=plsc.VectorSubcoreMesh(core_axis_name="c", subcore_axis_name="s"),
    scratch_types=[
        pltpu.VMEM((NH + 16,), jnp.int32),
        pltpu.VMEM((NH + 16,), jnp.int32),
        pltpu.VMEM((B_SC + 16,), jnp.int32),
        pltpu.VMEM((NH + 16,), jnp.float32),
        pltpu.VMEM((2 * CCW, RPC, N), jnp.float32),
        pltpu.VMEM((CCW, RPC, N), jnp.float32),
        pltpu.SemaphoreType.DMA((2 * CCW,)),
        pltpu.SemaphoreType.DMA((CCW,)),
        pltpu.SMEM((CCW,), jnp.int32),
    ],
)(_sc_body)


def _rank_order(h, nb):
    bidx = jnp.arange(nb, dtype=jnp.int32)
    less = h[None, :] < h[:, None]
    eq_lo = (h[None, :] == h[:, None]) & (bidx[None, :] < bidx[:, None])
    rank = jnp.sum(less | eq_lo, axis=1, dtype=jnp.int32)
    order = jnp.sum(bidx[:, None] * (rank[:, None] == bidx[None, :]),
                    axis=0, dtype=jnp.int32)
    return order


def _tc_scale_kernel(hours_ref, perm_ref, w_ref, imf_ref, out_ref):
    b = pl.program_id(0)
    h = hours_ref[b]
    wv = jnp.maximum(w_ref[h, 0], 0.0)
    out_ref[...] = imf_ref[...] * wv


def kernel(inputs, imf, weights):
    hours = (inputs[:, 0, 0, 1] * 24.0).astype(jnp.int32)
    harange = jnp.arange(NH, dtype=jnp.int32)

    htc = hours[:B_TC]
    perm = _rank_order(htc, B_TC)
    htc_sorted = jnp.take(htc, perm, axis=0)
    grid_spec = pltpu.PrefetchScalarGridSpec(
        num_scalar_prefetch=2,
        grid=(B_TC,),
        in_specs=[
            pl.BlockSpec((NH, 1), lambda b, hr, pr: (0, 0),
                         memory_space=pltpu.SMEM),
            pl.BlockSpec((1, N, N), lambda b, hr, pr: (hr[b], 0, 0)),
        ],
        out_specs=pl.BlockSpec((1, N, N), lambda b, hr, pr: (pr[b], 0, 0)),
    )
    tc_out = pl.pallas_call(
        _tc_scale_kernel,
        grid_spec=grid_spec,
        out_shape=jax.ShapeDtypeStruct((B_TC, N, N), jnp.float32),
    )(htc_sorted, perm, weights, imf)

    hsc = hours[B_TC:]
    order = _rank_order(hsc, B_SC)
    cnt = jnp.sum(hsc[None, :] == harange[:, None], axis=1, dtype=jnp.int32)
    start = (jnp.cumsum(cnt) - cnt).astype(jnp.int32)
    sc_out = _sc_call(
        jnp.pad(cnt, (0, 16)),
        jnp.pad(start, (0, 16)),
        jnp.pad(order, (0, 16)),
        jnp.pad(weights.reshape(NH), (0, 16)),
        imf,
    )
    return jnp.concatenate([tc_out, sc_out], axis=0)


# speedup vs baseline: 2.0817x; 2.0817x over previous
import functools

import jax
import jax.numpy as jnp
from jax import lax
from jax.experimental import pallas as pl
from jax.experimental.pallas import tpu as pltpu
from jax.experimental.pallas import tpu_sc as plsc

B, N = 64, 1024
NH = 24
RPC = 16
C = N // RPC
NC, NS = 2, 16
NW = NC * NS
CCW = C // NW


def _sc_body(cnt_hbm, start_hbm, order_hbm, w_hbm, imf_hbm, out_hbm,
             cnt_v, start_v, order_v, w_v, in_buf, out_buf,
             in_sems, out_sems, prev_smem):
    wid = lax.axis_index("s") * NC + lax.axis_index("c")

    def read_chunk(h, hh, cc):
        r0 = (cc * NW + wid) * RPC
        pltpu.async_copy(imf_hbm.at[h, pl.ds(r0, RPC), :],
                         in_buf.at[hh * CCW + cc], in_sems.at[hh * CCW + cc])

    for hh in range(2):
        for cc in range(CCW):
            read_chunk(hh, hh, cc)

    pltpu.sync_copy(cnt_hbm, cnt_v)
    pltpu.sync_copy(start_hbm, start_v)
    pltpu.sync_copy(order_hbm, order_v)
    pltpu.sync_copy(w_hbm, w_v)
    for cc in range(CCW):
        prev_smem[cc] = 0

    def _sget(ref, i):
        return ref[pl.ds(i, 16)][0]

    @pl.loop(0, NH, step=2)
    def _hloop(g):
        for hh in range(2):
            h = g + hh
            wv = jnp.maximum(_sget(w_v, h), 0.0)
            cnt = _sget(cnt_v, h)
            st = _sget(start_v, h)
            for cc in range(CCW):
                slot = hh * CCW + cc
                r0 = (cc * NW + wid) * RPC

                pltpu.make_async_copy(imf_hbm.at[0, pl.ds(0, RPC), :],
                                      in_buf.at[slot], in_sems.at[slot]).wait()

                prev = prev_smem[cc]

                @pl.loop(0, prev)
                def _drain(j):
                    pltpu.make_async_copy(out_buf.at[cc],
                                          out_hbm.at[0, pl.ds(0, RPC), :],
                                          out_sems.at[cc]).wait()

                for r in range(RPC):

                    @plsc.parallel_loop(0, N, step=16, unroll=16)
                    def _scale(j):
                        sl = pl.ds(j, 16)
                        out_buf[cc, r, sl] = in_buf[slot, r, sl] * wv

                @pl.when(h + 2 < NH)
                def _prefetch():
                    read_chunk(h + 2, hh, cc)

                @pl.loop(0, cnt)
                def _writes(j):
                    b = _sget(order_v, st + j)
                    pltpu.async_copy(out_buf.at[cc],
                                     out_hbm.at[b, pl.ds(r0, RPC), :],
                                     out_sems.at[cc])

                prev_smem[cc] = cnt

    for cc in range(CCW):
        prev = prev_smem[cc]

        @pl.loop(0, prev)
        def _final_drain(j):
            pltpu.make_async_copy(out_buf.at[cc],
                                  out_hbm.at[0, pl.ds(0, RPC), :],
                                  out_sems.at[cc]).wait()


_sc_call = functools.partial(
    pl.kernel,
    out_type=jax.ShapeDtypeStruct((B, N, N), jnp.float32),
    mesh=plsc.VectorSubcoreMesh(core_axis_name="c", subcore_axis_name="s"),
    scratch_types=[
        pltpu.VMEM((NH + 16,), jnp.int32),
        pltpu.VMEM((NH + 16,), jnp.int32),
        pltpu.VMEM((B + 16,), jnp.int32),
        pltpu.VMEM((NH + 16,), jnp.float32),
        pltpu.VMEM((2 * CCW, RPC, N), jnp.float32),
        pltpu.VMEM((CCW, RPC, N), jnp.float32),
        pltpu.SemaphoreType.DMA((2 * CCW,)),
        pltpu.SemaphoreType.DMA((CCW,)),
        pltpu.SMEM((CCW,), jnp.int32),
    ],
)(_sc_body)


def kernel(inputs, imf, weights):
    hours = (inputs[:, 0, 0, 1] * 24.0).astype(jnp.int32)
    bidx = jnp.arange(B, dtype=jnp.int32)
    less = hours[None, :] < hours[:, None]
    eq_lo = (hours[None, :] == hours[:, None]) & (bidx[None, :] < bidx[:, None])
    rank = jnp.sum(less | eq_lo, axis=1, dtype=jnp.int32)
    order = jnp.sum(bidx[:, None] * (rank[:, None] == bidx[None, :]),
                    axis=0, dtype=jnp.int32)
    harange = jnp.arange(NH, dtype=jnp.int32)
    cnt = jnp.sum(hours[None, :] == harange[:, None], axis=1,
                  dtype=jnp.int32)
    start = (jnp.cumsum(cnt) - cnt).astype(jnp.int32)
    return _sc_call(
        jnp.pad(cnt, (0, 16)),
        jnp.pad(start, (0, 16)),
        jnp.pad(order, (0, 16)),
        jnp.pad(weights.reshape(NH), (0, 16)),
        imf,
    )
